# baseline (speedup 1.0000x reference)
import functools
import math

import jax
import jax.numpy as jnp
from jax import lax
from jax.experimental import pallas as pl
from jax.experimental.pallas import tpu as pltpu
from jax.experimental.pallas import tpu_sc as plsc

EMB = 32
SCALE = math.sqrt(EMB)
LANES = 16
IW = 128
CR = 4

_info = plsc.get_sparse_core_info()
_NC, _NS = _info.num_cores, _info.num_subcores
NW = _NC * _NS

NBUF = 4
NOBUF = 2


@jax.jit
def _embed(tok_lin, table):
    R = tok_lin.shape[0]
    NS_TOT = 8 * (R // 1024)
    rpw = R // NW
    nchunk = rpw // CR
    ngroup = nchunk // NBUF

    mesh = plsc.VectorSubcoreMesh(core_axis_name="c", subcore_axis_name="s")

    @functools.partial(
        pl.kernel,
        mesh=mesh,
        compiler_params=pltpu.CompilerParams(
            use_tc_tiling_on_sc=False, needs_layout_passes=False),
        out_type=jax.ShapeDtypeStruct((NS_TOT, 4, 128, 8, IW), jnp.float32),
        scratch_types=[
            pltpu.VMEM((NBUF, CR, IW), jnp.int32),
            pltpu.VMEM((NBUF, CR, IW, EMB), jnp.float32),
            pltpu.VMEM((NOBUF, CR, 4, 8, IW), jnp.float32),
        ]
        + [pltpu.SemaphoreType.DMA] * (3 * NBUF + NOBUF),
    )
    def body(tok_hbm, table_hbm, out_hbm, idx_v, rows_v, trans_v, *sems):
        gsem = sems[:NBUF]
        isem = sems[NBUF:2 * NBUF]
        osem = sems[2 * NBUF:2 * NBUF + NOBUF]
        wid = lax.axis_index("s") * _NC + lax.axis_index("c")
        base_r = wid * rpw
        iot = jax.lax.iota(jnp.int32, LANES)

        def idx_copy(i, b):
            return pltpu.make_async_copy(
                tok_hbm.at[pl.ds(base_r + i * CR, CR)], idx_v.at[b], isem[b])

        def issue_gather(b):
            for j in range(CR):
                pltpu.async_copy(table_hbm.at[idx_v.at[b, j]],
                                 rows_v.at[b, j], gsem[b])

        def wait_gather(b):
            for j in range(CR):
                pltpu.make_async_copy(table_hbm.at[idx_v.at[b, j]],
                                      rows_v.at[b, j], gsem[b]).wait()

        def transpose_scale(b, c):
            for j in range(CR):
                def mstep(m, _):
                    e = m & (EMB - 1)
                    bi0 = (m >> 5) << 4
                    ev = jnp.full((LANES,), e, jnp.int32)
                    g = plsc.load_gather(rows_v.at[b, j], [bi0 + iot, ev])
                    trans_v[c, j, e >> 3, e & 7, pl.ds(bi0, LANES)] = g * SCALE
                    return ()

                lax.fori_loop(0, (IW // LANES) * EMB, mstep, (), unroll=4)

        def out_copy(i, c, j):
            r = base_r + i * CR + j
            s = ((r >> 10) << 3) | (r & 7)
            bb = (r >> 3) & 127
            return pltpu.make_async_copy(
                trans_v.at[c, j], out_hbm.at[s, :, bb], osem[c])

        for b in range(NBUF):
            idx_copy(b, b).start()
        for b in range(NBUF):
            idx_copy(b, b).wait()
            issue_gather(b)

        def group(g, _):
            for b in range(NBUF):
                i = g * NBUF + b
                c = b % NOBUF
                wait_gather(b)

                @pl.when(i + NBUF < nchunk)
                def _prefetch_idx():
                    idx_copy(i + NBUF, b).start()

                @pl.when(i >= NOBUF)
                def _drain_out():
                    for j in range(CR):
                        out_copy(i - NOBUF, c, j).wait()

                transpose_scale(b, c)
                for j in range(CR):
                    out_copy(i, c, j).start()

                @pl.when(i + NBUF < nchunk)
                def _issue_next():
                    idx_copy(i + NBUF, b).wait()
                    issue_gather(b)

            return ()

        lax.fori_loop(0, ngroup, group, ())

        for k in range(NOBUF):
            i = nchunk - NOBUF + k
            for j in range(CR):
                out_copy(i, i % NOBUF, j).wait()

    return body(tok_lin, table)


def kernel(tokens, table):
    NB, NS_TOT = tokens.shape
    t = tokens.astype(jnp.int32).reshape(128, NB // 128, NS_TOT // 8, 8)
    tok_lin = t.transpose(2, 0, 3, 1).reshape(NB * NS_TOT // IW, IW)
    out5d = _embed(tok_lin, table)
    out = out5d.transpose(2, 4, 0, 1, 3).reshape(NB, NS_TOT, EMB)
    return out


# speedup vs baseline: 1.7896x; 1.7896x over previous
import functools
import math

import jax
import jax.numpy as jnp
from jax import lax
from jax.experimental import pallas as pl
from jax.experimental.pallas import tpu as pltpu
from jax.experimental.pallas import tpu_sc as plsc

EMB = 32
SCALE = math.sqrt(EMB)
LANES = 16
IW = 128
CR = 4

_info = plsc.get_sparse_core_info()
_NC, _NS = _info.num_cores, _info.num_subcores
NW = _NC * _NS

NBUF = 4
NOBUF = 2


@jax.jit
def _embed(tok_lin, table):
    R = tok_lin.shape[0]
    NS_TOT = 8 * (R // 1024)
    rpw = R // NW
    nchunk = rpw // CR
    ngroup = nchunk // NBUF

    mesh = plsc.VectorSubcoreMesh(core_axis_name="c", subcore_axis_name="s")

    @functools.partial(
        pl.kernel,
        mesh=mesh,
        compiler_params=pltpu.CompilerParams(
            use_tc_tiling_on_sc=False, needs_layout_passes=False),
        out_type=jax.ShapeDtypeStruct((NS_TOT, 4, 128, 8 * IW), jnp.float32),
        scratch_types=[
            pltpu.VMEM((NBUF, CR, IW), jnp.int32),
            pltpu.VMEM((NBUF, CR, IW, EMB), jnp.float32),
            pltpu.VMEM((NOBUF, CR, 4, 8 * IW), jnp.float32),
        ]
        + [pltpu.SemaphoreType.DMA] * (3 * NBUF + NOBUF),
    )
    def body(tok_hbm, table_hbm, out_hbm, idx_v, rows_v, trans_v, *sems):
        gsem = sems[:NBUF]
        isem = sems[NBUF:2 * NBUF]
        osem = sems[2 * NBUF:2 * NBUF + NOBUF]
        wid = lax.axis_index("s") * _NC + lax.axis_index("c")
        base_r = wid * rpw
        iot = jax.lax.iota(jnp.int32, LANES)

        def idx_copy(i, b):
            return pltpu.make_async_copy(
                tok_hbm.at[pl.ds(base_r + i * CR, CR)], idx_v.at[b], isem[b])

        def issue_gather(b):
            for j in range(CR):
                pltpu.async_copy(table_hbm.at[idx_v.at[b, j]],
                                 rows_v.at[b, j], gsem[b])

        def wait_gather(b):
            for j in range(CR):
                pltpu.make_async_copy(table_hbm.at[idx_v.at[b, j]],
                                      rows_v.at[b, j], gsem[b]).wait()

        def transpose_scale(b, c):
            for j in range(CR):
                def mstep(m, _):
                    e0 = m & (EMB - 1)
                    bi0 = (m >> 5) << 4
                    bi_idx = bi0 + iot
                    e_idx = (e0 + iot) & (EMB - 1)
                    g = plsc.load_gather(rows_v.at[b, j], [bi_idx, e_idx])
                    plsc.store_scatter(
                        trans_v.at[c, j],
                        [e_idx >> 3, ((e_idx & 7) << 7) + bi_idx], g * SCALE)
                    return ()

                lax.fori_loop(0, (IW // LANES) * EMB, mstep, (), unroll=4)

        def out_copy(i, c, j):
            r = base_r + i * CR + j
            s = ((r >> 10) << 3) | (r & 7)
            bb = (r >> 3) & 127
            return pltpu.make_async_copy(
                trans_v.at[c, j], out_hbm.at[s, :, bb], osem[c])

        for b in range(NBUF):
            idx_copy(b, b).start()
        for b in range(NBUF):
            idx_copy(b, b).wait()
            issue_gather(b)

        def group(g, _):
            for b in range(NBUF):
                i = g * NBUF + b
                c = b % NOBUF
                wait_gather(b)

                @pl.when(i + NBUF < nchunk)
                def _prefetch_idx():
                    idx_copy(i + NBUF, b).start()

                @pl.when(i >= NOBUF)
                def _drain_out():
                    for j in range(CR):
                        out_copy(i - NOBUF, c, j).wait()

                transpose_scale(b, c)
                for j in range(CR):
                    out_copy(i, c, j).start()

                @pl.when(i + NBUF < nchunk)
                def _issue_next():
                    idx_copy(i + NBUF, b).wait()
                    issue_gather(b)

            return ()

        lax.fori_loop(0, ngroup, group, ())

        for k in range(NOBUF):
            i = nchunk - NOBUF + k
            for j in range(CR):
                out_copy(i, i % NOBUF, j).wait()

    return body(tok_lin, table)


def kernel(tokens, table):
    NB, NS_TOT = tokens.shape
    t = tokens.astype(jnp.int32).reshape(128, NB // 128, NS_TOT // 8, 8)
    tok_lin = t.transpose(2, 0, 3, 1).reshape(NB * NS_TOT // IW, IW)
    out4d = _embed(tok_lin, table)
    out5d = out4d.reshape(NS_TOT, 4, 128, 8, IW)
    out = out5d.transpose(2, 4, 0, 1, 3).reshape(NB, NS_TOT, EMB)
    return out
